# baseline (speedup 1.0000x reference)
import functools

import jax
import jax.numpy as jnp
from jax import lax
from jax.experimental import pallas as pl
from jax.experimental.pallas import tpu as pltpu
from jax.experimental.pallas import tpu_sc as plsc

L = 16


def _make_sc_kernel(B, H, W, C):
    mesh = plsc.VectorSubcoreMesh(core_axis_name="c", subcore_axis_name="s")
    n_vec = C // L

    @functools.partial(
        pl.kernel,
        mesh=mesh,
        out_type=jax.ShapeDtypeStruct((B, H, W, C), jnp.float32),
        scratch_types=[
            pltpu.VMEM((2, W, C), jnp.float32),
            pltpu.VMEM((2, W, C), jnp.float32),
            pltpu.VMEM((W, C), jnp.float32),
            pltpu.VMEM((C,), jnp.float32),
            pltpu.SemaphoreType.DMA,
            pltpu.SemaphoreType.DMA,
            pltpu.SemaphoreType.DMA,
            pltpu.SemaphoreType.DMA,
        ],
    )
    def sc_kernel(x_hbm, row_hbm, col_hbm, out_hbm,
                  ibuf, obuf, pos, rowv, isem0, isem1, osem0, osem1):
        isems = (isem0, isem1)
        osems = (osem0, osem1)
        h = lax.axis_index("s") * 2 + lax.axis_index("c")

        pltpu.sync_copy(col_hbm, pos)
        pltpu.sync_copy(row_hbm.at[h], rowv)

        def pos_body(w, carry):
            for j in range(n_vec):
                sl = pl.ds(j * L, L)
                pos[w, sl] = pos[w, sl] + rowv[sl]
            return carry

        lax.fori_loop(0, W, pos_body, 0)

        pltpu.async_copy(x_hbm.at[0, h], ibuf.at[0], isems[0])
        pltpu.async_copy(x_hbm.at[1, h], ibuf.at[1], isems[1])

        def batch_pair(bb, carry):
            for slot in range(2):
                b = 2 * bb + slot
                pltpu.make_async_copy(
                    x_hbm.at[b, h], ibuf.at[slot], isems[slot]).wait()

                @pl.when(bb >= 1)
                def _wait_out():
                    pltpu.make_async_copy(
                        obuf.at[slot], out_hbm.at[b - 2, h], osems[slot]).wait()

                def add_body(w, c2):
                    for j in range(n_vec):
                        sl = pl.ds(j * L, L)
                        obuf[slot, w, sl] = ibuf[slot, w, sl] + pos[w, sl]
                    return c2

                lax.fori_loop(0, W, add_body, 0)

                pltpu.async_copy(obuf.at[slot], out_hbm.at[b, h], osems[slot])

                @pl.when(b + 2 < B)
                def _prefetch():
                    pltpu.async_copy(
                        x_hbm.at[b + 2, h], ibuf.at[slot], isems[slot])
            return carry

        lax.fori_loop(0, B // 2, batch_pair, 0)

        pltpu.make_async_copy(
            obuf.at[0], out_hbm.at[B - 2, h], osems[0]).wait()
        pltpu.make_async_copy(
            obuf.at[1], out_hbm.at[B - 1, h], osems[1]).wait()

    return sc_kernel


def kernel(x, row_emb, col_emb):
    b, h, w, c = x.shape
    return _make_sc_kernel(b, h, w, c)(x, row_emb, col_emb)


# speedup vs baseline: 1.8898x; 1.8898x over previous
import functools

import jax
import jax.numpy as jnp
from jax import lax
from jax.experimental import pallas as pl
from jax.experimental.pallas import tpu as pltpu
from jax.experimental.pallas import tpu_sc as plsc

L = 16
NSLOTS = 4


def _make_sc_kernel(B, H, W, C):
    mesh = plsc.VectorSubcoreMesh(core_axis_name="c", subcore_axis_name="s")
    n_vec = C // L

    @functools.partial(
        pl.kernel,
        mesh=mesh,
        out_type=jax.ShapeDtypeStruct((B, H, W, C), jnp.float32),
        scratch_types=[
            pltpu.VMEM((NSLOTS, W, C), jnp.float32),
            pltpu.VMEM((W, C), jnp.float32),
            pltpu.VMEM((C,), jnp.float32),
        ]
        + [pltpu.SemaphoreType.DMA] * (2 * NSLOTS),
    )
    def sc_kernel(x_hbm, row_hbm, col_hbm, out_hbm, buf, pos, rowv, *sems):
        isems = sems[:NSLOTS]
        osems = sems[NSLOTS:]
        h = lax.axis_index("s") * 2 + lax.axis_index("c")

        pltpu.sync_copy(col_hbm, pos)
        pltpu.sync_copy(row_hbm.at[h], rowv)

        @plsc.parallel_loop(0, W, unroll=2)
        def _pos_body(w):
            for j in range(n_vec):
                sl = pl.ds(j * L, L)
                plsc.addupdate(pos.at[w, sl], rowv[sl])

        for s in range(NSLOTS):
            pltpu.async_copy(x_hbm.at[s, h], buf.at[s], isems[s])

        def group(g, carry):
            for s in range(NSLOTS):
                b = NSLOTS * g + s
                pltpu.make_async_copy(
                    x_hbm.at[b, h], buf.at[s], isems[s]).wait()

                @plsc.parallel_loop(0, W, unroll=2)
                def _add_body(w):
                    for j in range(n_vec):
                        sl = pl.ds(j * L, L)
                        plsc.addupdate(buf.at[s, w, sl], pos[w, sl])

                pltpu.async_copy(buf.at[s], out_hbm.at[b, h], osems[s])

                sp = (s - 1) % NSLOTS
                bp = b - 1

                def retire_and_prefetch():
                    pltpu.make_async_copy(
                        buf.at[sp], out_hbm.at[bp, h], osems[sp]).wait()

                    @pl.when(bp + NSLOTS < B)
                    def _prefetch():
                        pltpu.async_copy(
                            x_hbm.at[bp + NSLOTS, h], buf.at[sp], isems[sp])

                if s == 0:
                    pl.when(g > 0)(retire_and_prefetch)
                else:
                    retire_and_prefetch()
            return carry

        lax.fori_loop(0, B // NSLOTS, group, 0)

        pltpu.make_async_copy(
            buf.at[NSLOTS - 1], out_hbm.at[B - 1, h],
            osems[NSLOTS - 1]).wait()

    return sc_kernel


def kernel(x, row_emb, col_emb):
    b, h, w, c = x.shape
    return _make_sc_kernel(b, h, w, c)(x, row_emb, col_emb)
